# baseline (speedup 1.0000x reference)
import functools

import jax
import jax.numpy as jnp
from jax import lax
from jax.experimental import pallas as pl
from jax.experimental.pallas import tpu as pltpu

_EPS = 1e-5
_DEPTH = 4


def _fused_body(w_ref, gamma_ref, beta_ref, x_hbm, o_hbm,
                xbf_ref, g_ref, s_ref, xin_ref, obuf_ref, in_sem, out_sem,
                *, n, m_true):
    c_in = w_ref.shape[1]

    def cp_in(i):
        return pltpu.make_async_copy(
            x_hbm.at[i], xin_ref.at[i % _DEPTH], in_sem.at[i % _DEPTH])

    def cp_out(i):
        return pltpu.make_async_copy(
            obuf_ref.at[i % _DEPTH], o_hbm.at[i], out_sem.at[i % _DEPTH])

    g_ref[...] = jnp.zeros_like(g_ref)
    s_ref[...] = jnp.zeros_like(s_ref)
    for i in range(min(_DEPTH, n)):
        cp_in(i).start()
    for i in range(n):
        cp_in(i).wait()
        x = xin_ref[i % _DEPTH]
        xb = x.astype(jnp.bfloat16)
        xbf_ref[i] = xb
        g_ref[...] += lax.dot_general(
            xb, xb, (((1,), (1,)), ((), ())),
            preferred_element_type=jnp.float32)
        s_ref[...] += jnp.sum(x, axis=1, keepdims=True)
        if i + _DEPTH < n:
            cp_in(i + _DEPTH).start()

    w = w_ref[...].astype(jnp.float32)
    ws = jnp.dot(w, jnp.broadcast_to(s_ref[...], (c_in, c_in)),
                 preferred_element_type=jnp.float32)[:, :1]
    mean = ws / m_true
    wg = jnp.dot(w, g_ref[...], preferred_element_type=jnp.float32)
    e_y2 = jnp.sum(wg * w, axis=1, keepdims=True) / m_true
    var = jnp.maximum(e_y2 - mean * mean, 0.0)
    inv = lax.rsqrt(var + _EPS)
    scale = gamma_ref[...] * inv
    shift = beta_ref[...] - mean * scale
    wf = (scale * w).astype(jnp.bfloat16)

    for i in range(n):
        if i >= _DEPTH:
            cp_out(i - _DEPTH).wait()
        y = jnp.dot(wf, xbf_ref[i],
                    preferred_element_type=jnp.float32)
        obuf_ref[i % _DEPTH] = jnp.maximum(y + shift, 0.0)
        cp_out(i).start()
    for i in range(max(0, n - _DEPTH), n):
        cp_out(i).wait()


def kernel(x_nchw, weight, gamma, beta):
    N, C_in, H, W = x_nchw.shape
    C_out = weight.shape[0]
    HW = H * W
    M = float(N * HW)
    x3d = x_nchw.reshape(N, C_in, HW)
    g2 = gamma.reshape(C_out, 1).astype(jnp.float32)
    b2 = beta.reshape(C_out, 1).astype(jnp.float32)

    body = functools.partial(_fused_body, n=N, m_true=M)

    out3d = pl.pallas_call(
        body,
        out_shape=jax.ShapeDtypeStruct((N, C_out, HW), jnp.float32),
        in_specs=[
            pl.BlockSpec((C_out, C_in), lambda i: (0, 0)),
            pl.BlockSpec((C_out, 1), lambda i: (0, 0)),
            pl.BlockSpec((C_out, 1), lambda i: (0, 0)),
            pl.BlockSpec(memory_space=pl.ANY),
        ],
        out_specs=pl.BlockSpec(memory_space=pl.ANY),
        scratch_shapes=[
            pltpu.VMEM((N, C_in, HW), jnp.bfloat16),
            pltpu.VMEM((C_in, C_in), jnp.float32),
            pltpu.VMEM((C_in, 1), jnp.float32),
            pltpu.VMEM((_DEPTH, C_in, HW), jnp.float32),
            pltpu.VMEM((_DEPTH, C_out, HW), jnp.float32),
            pltpu.SemaphoreType.DMA((_DEPTH,)),
            pltpu.SemaphoreType.DMA((_DEPTH,)),
        ],
        compiler_params=pltpu.CompilerParams(
            dimension_semantics=("arbitrary",)),
        cost_estimate=pl.CostEstimate(
            flops=2 * N * HW * C_in * (C_in + C_out), transcendentals=C_out,
            bytes_accessed=4 * N * HW * (C_in + C_out)),
        grid=(1,),
    )(weight, g2, b2, x3d)

    return out3d.reshape(N, C_out, H, W)
